# baseline (speedup 1.0000x reference)
import functools
import math

import jax
import jax.numpy as jnp
from jax import lax
from jax.experimental import pallas as pl
from jax.experimental.pallas import tpu as pltpu
from jax.experimental.pallas import tpu_sc as plsc

D_MODEL = 64
SCALE = math.sqrt(D_MODEL)

NC, NS, L = 2, 16, 16
NW = NC * NS

CHUNK = 128


@functools.lru_cache(maxsize=None)
def _make_kernel(n_chunks: int, D: int):
    mesh = plsc.VectorSubcoreMesh(core_axis_name="c", subcore_axis_name="s")

    @functools.partial(
        pl.kernel,
        mesh=mesh,
        out_type=jax.ShapeDtypeStruct((NW, n_chunks, CHUNK, D), jnp.float32),
        scratch_types=[
            pltpu.VMEM((n_chunks, CHUNK), jnp.int32),
            pltpu.VMEM((CHUNK, D), jnp.float32),
            pltpu.SemaphoreType.DMA,
        ],
        compiler_params=pltpu.CompilerParams(use_tc_tiling_on_sc=False),
    )
    def k(idx_hbm, table_hbm, out_hbm, idx_v, rows_v, sem):
        wid = lax.axis_index("s") * NC + lax.axis_index("c")
        pltpu.sync_copy(idx_hbm.at[wid], idx_v)

        def chunk_body(j, carry):
            pltpu.async_copy(table_hbm.at[idx_v.at[j]], rows_v, sem).wait()

            def row_body(r, c2):
                for c in range(D // L):
                    sl = pl.ds(c * L, L)
                    rows_v[r, sl] = rows_v[r, sl] * SCALE
                return c2

            lax.fori_loop(0, CHUNK, row_body, 0)
            pltpu.sync_copy(rows_v, out_hbm.at[wid, j])
            return carry

        lax.fori_loop(0, n_chunks, chunk_body, 0)

    return k


def kernel(x, table):
    B = x.size
    D = table.shape[1]
    n_chunks = B // (NW * CHUNK)
    idx = jnp.reshape(x.astype(jnp.int32), (NW, n_chunks, CHUNK))
    out = _make_kernel(n_chunks, D)(idx, table)
    return jnp.reshape(out, x.shape + (D,))
